# baseline (speedup 1.0000x reference)
import functools

import jax
import jax.numpy as jnp
from jax.experimental import pallas as pl
from jax.experimental.pallas import tpu as pltpu


def _prelu(x, a):
    return jnp.where(x >= 0, x, a * x)


_DN_T = (((1,), (1,)), ((), ()))


def _mega_body(aug1_hbm, aug2_hbm, adj_hbm, seq1_ref, seq2_ref, w_ref,
               b_ref, a_ref, wb_ref, bb_ref,
               o1_ref, o2_ref,
               buf, s_ref, m1_ref, m3_ref, v_ref, sems,
               *, h, bm, nb, inv_n):
    nsteps = 3 * nb

    def issue(t, slot):
        r = jax.lax.rem(t, nb) * bm

        @pl.when(t < nb)
        def _():
            pltpu.make_async_copy(
                aug1_hbm.at[0, pl.ds(r, bm), :], buf.at[slot],
                sems.at[slot]).start()

        @pl.when(jnp.logical_and(t >= nb, t < 2 * nb))
        def _():
            pltpu.make_async_copy(
                aug2_hbm.at[0, pl.ds(r, bm), :], buf.at[slot],
                sems.at[slot]).start()

        @pl.when(t >= 2 * nb)
        def _():
            pltpu.make_async_copy(
                adj_hbm.at[0, pl.ds(r, bm), :], buf.at[slot],
                sems.at[slot]).start()

    issue(0, 0)
    issue(1, 1)

    w = w_ref[...].astype(jnp.bfloat16)
    s_ref[:, :h] = jax.lax.dot_general(
        seq1_ref[0].astype(jnp.bfloat16), w, _DN_T,
        preferred_element_type=jnp.float32).astype(jnp.bfloat16)
    s_ref[:, h:] = jax.lax.dot_general(
        seq2_ref[0].astype(jnp.bfloat16), w, _DN_T,
        preferred_element_type=jnp.float32).astype(jnp.bfloat16)
    m1_ref[...] = jnp.zeros_like(m1_ref)
    m3_ref[...] = jnp.zeros_like(m3_ref)

    a = a_ref[0, 0]
    b = b_ref[...]

    def one_step(t, slot, compute):
        pltpu.make_async_copy(
            aug1_hbm.at[0, pl.ds(0, bm), :], buf.at[slot],
            sems.at[slot]).wait()
        x = buf[slot].astype(jnp.bfloat16)
        compute(t, x)

        @pl.when(t + 2 < nsteps)
        def _next():
            issue(t + 2, slot)

    def run_phase(start, compute):
        p = start % 2

        def pair(j, carry):
            t = start + 2 * j
            one_step(t, p, compute)
            one_step(t + 1, 1 - p, compute)
            return carry

        jax.lax.fori_loop(0, nb // 2, pair, 0)
        if nb % 2:
            t_last = start + nb - 1
            one_step(t_last, t_last % 2, compute)

    def aug1_compute(t, x):
        p = _prelu(jnp.dot(x, s_ref[:, :h],
                           preferred_element_type=jnp.float32) + b, a)
        m1_ref[...] += jnp.sum(p, axis=0, keepdims=True)

    def aug2_compute(t, x):
        p = _prelu(jnp.dot(x, s_ref[:, :h],
                           preferred_element_type=jnp.float32) + b, a)
        m3_ref[...] += jnp.sum(p, axis=0, keepdims=True)

    def adj_compute(t, x):
        acc = jnp.dot(x, s_ref[...],
                      preferred_element_type=jnp.float32)
        h0 = _prelu(acc[:, :h] + b, a).astype(jnp.bfloat16)
        h2 = _prelu(acc[:, h:] + b, a).astype(jnp.bfloat16)
        v = v_ref[...].astype(jnp.bfloat16)
        two_bb = 2.0 * bb_ref[0, 0]
        row = t - 2 * nb
        o1_ref[pl.ds(row, 1), :] = jax.lax.dot_general(
            v, h0, _DN_T, preferred_element_type=jnp.float32) + two_bb
        o2_ref[pl.ds(row, 1), :] = jax.lax.dot_general(
            v, h2, _DN_T, preferred_element_type=jnp.float32) + two_bb

    run_phase(0, aug1_compute)
    run_phase(nb, aug2_compute)
    c = (jax.nn.sigmoid(m1_ref[...] * inv_n)
         + jax.nn.sigmoid(m3_ref[...] * inv_n))
    v_ref[...] = jax.lax.dot_general(
        c, wb_ref[0], _DN_T, preferred_element_type=jnp.float32)
    run_phase(2 * nb, adj_compute)


def kernel(seq1, seq2, seq3, seq4, adj, aug_adj1, aug_adj2,
           W_gcn, b_gcn, prelu_a, W_bil, b_bil):
    del seq3, seq4
    _, n, n_in = seq1.shape
    h = W_gcn.shape[0]
    bm = 400 if n % 400 == 0 else (8 if n % 8 == 0 else 1)
    nb = n // bm

    b2 = b_gcn.reshape(1, h)
    a2 = prelu_a.reshape(1, 1)
    hbm_spec = pl.BlockSpec(memory_space=pltpu.MemorySpace.HBM)

    o1, o2 = pl.pallas_call(
        functools.partial(_mega_body, h=h, bm=bm, nb=nb,
                          inv_n=float(1.0 / n)),
        in_specs=[
            hbm_spec,
            hbm_spec,
            hbm_spec,
            pl.BlockSpec((1, n, n_in), lambda: (0, 0, 0)),
            pl.BlockSpec((1, n, n_in), lambda: (0, 0, 0)),
            pl.BlockSpec((h, n_in), lambda: (0, 0)),
            pl.BlockSpec((1, h), lambda: (0, 0)),
            pl.BlockSpec((1, 1), lambda: (0, 0)),
            pl.BlockSpec((1, h, h), lambda: (0, 0, 0)),
            pl.BlockSpec((1, 1), lambda: (0, 0)),
        ],
        out_specs=[
            pl.BlockSpec((nb, bm), lambda: (0, 0)),
            pl.BlockSpec((nb, bm), lambda: (0, 0)),
        ],
        out_shape=[
            jax.ShapeDtypeStruct((nb, bm), jnp.float32),
            jax.ShapeDtypeStruct((nb, bm), jnp.float32),
        ],
        scratch_shapes=[
            pltpu.VMEM((2, bm, n), jnp.float32),
            pltpu.VMEM((n, 2 * h), jnp.bfloat16),
            pltpu.VMEM((1, h), jnp.float32),
            pltpu.VMEM((1, h), jnp.float32),
            pltpu.VMEM((1, h), jnp.float32),
            pltpu.SemaphoreType.DMA((2,)),
        ],
    )(aug_adj1, aug_adj2, adj, seq1, seq2, W_gcn, b2, a2,
      W_bil, b_bil.reshape(1, 1))

    return jnp.concatenate([o1.reshape(1, n), o2.reshape(1, n)], axis=1)


# speedup vs baseline: 1.0486x; 1.0486x over previous
import functools

import jax
import jax.numpy as jnp
from jax.experimental import pallas as pl
from jax.experimental.pallas import tpu as pltpu


def _prelu(x, a):
    return jnp.where(x >= 0, x, a * x)


_DN_T = (((1,), (1,)), ((), ()))


def _mega_body(aug1_hbm, aug2_hbm, adj_hbm, seq1_ref, seq2_ref, w_ref,
               b_ref, a_ref, wb_ref, bb_ref,
               o1_ref, o2_ref,
               buf, s_ref, m1_ref, m3_ref, v_ref, sems,
               *, h, bm, nb, inv_n):
    nsteps = 3 * nb

    def issue(t, slot):
        r = jax.lax.rem(t, nb) * bm

        @pl.when(t < nb)
        def _():
            pltpu.make_async_copy(
                aug1_hbm.at[0, pl.ds(r, bm), :], buf.at[slot],
                sems.at[slot]).start()

        @pl.when(jnp.logical_and(t >= nb, t < 2 * nb))
        def _():
            pltpu.make_async_copy(
                aug2_hbm.at[0, pl.ds(r, bm), :], buf.at[slot],
                sems.at[slot]).start()

        @pl.when(t >= 2 * nb)
        def _():
            pltpu.make_async_copy(
                adj_hbm.at[0, pl.ds(r, bm), :], buf.at[slot],
                sems.at[slot]).start()

    issue(0, 0)
    issue(1, 1)

    w = w_ref[...].astype(jnp.bfloat16)
    s_ref[:, :h] = jax.lax.dot_general(
        seq1_ref[0].astype(jnp.bfloat16), w, _DN_T,
        preferred_element_type=jnp.float32).astype(jnp.bfloat16)
    s_ref[:, h:] = jax.lax.dot_general(
        seq2_ref[0].astype(jnp.bfloat16), w, _DN_T,
        preferred_element_type=jnp.float32).astype(jnp.bfloat16)
    m1_ref[...] = jnp.zeros_like(m1_ref)
    m3_ref[...] = jnp.zeros_like(m3_ref)

    a = a_ref[0, 0]
    b = b_ref[...]

    def fetch(t):
        slot = jax.lax.rem(t, 2)
        pltpu.make_async_copy(
            aug1_hbm.at[0, pl.ds(0, bm), :], buf.at[slot],
            sems.at[slot]).wait()
        return slot, buf[slot].astype(jnp.bfloat16)

    def aug_step(t, m_ref):
        slot, x = fetch(t)
        s1 = s_ref[:, :h]
        p = _prelu(jnp.dot(x, s1, preferred_element_type=jnp.float32)
                   + b, a)
        m_ref[...] += jnp.sum(p, axis=0, keepdims=True)

        @pl.when(t + 2 < nsteps)
        def _next():
            issue(t + 2, slot)

    def step_a(t, carry):
        aug_step(t, m1_ref)
        return carry

    def step_b(t, carry):
        aug_step(t, m3_ref)
        return carry

    def step_c(t, carry):
        slot, x = fetch(t)
        acc = jnp.dot(x, s_ref[...],
                      preferred_element_type=jnp.float32)
        h0 = _prelu(acc[:, :h] + b, a).astype(jnp.bfloat16)
        h2 = _prelu(acc[:, h:] + b, a).astype(jnp.bfloat16)
        v = v_ref[...].astype(jnp.bfloat16)
        two_bb = 2.0 * bb_ref[0, 0]
        row = t - 2 * nb
        o1_ref[pl.ds(row, 1), :] = jax.lax.dot_general(
            v, h0, _DN_T, preferred_element_type=jnp.float32) + two_bb
        o2_ref[pl.ds(row, 1), :] = jax.lax.dot_general(
            v, h2, _DN_T, preferred_element_type=jnp.float32) + two_bb

        @pl.when(t + 2 < nsteps)
        def _next():
            issue(t + 2, slot)

        return carry

    jax.lax.fori_loop(0, nb, step_a, 0)
    jax.lax.fori_loop(nb, 2 * nb, step_b, 0)
    c = (jax.nn.sigmoid(m1_ref[...] * inv_n)
         + jax.nn.sigmoid(m3_ref[...] * inv_n))
    v_ref[...] = jax.lax.dot_general(
        c, wb_ref[0], _DN_T, preferred_element_type=jnp.float32)
    jax.lax.fori_loop(2 * nb, 3 * nb, step_c, 0)


def kernel(seq1, seq2, seq3, seq4, adj, aug_adj1, aug_adj2,
           W_gcn, b_gcn, prelu_a, W_bil, b_bil):
    del seq3, seq4
    _, n, n_in = seq1.shape
    h = W_gcn.shape[0]
    bm = 400 if n % 400 == 0 else (8 if n % 8 == 0 else 1)
    nb = n // bm

    b2 = b_gcn.reshape(1, h)
    a2 = prelu_a.reshape(1, 1)
    hbm_spec = pl.BlockSpec(memory_space=pltpu.MemorySpace.HBM)

    o1, o2 = pl.pallas_call(
        functools.partial(_mega_body, h=h, bm=bm, nb=nb,
                          inv_n=float(1.0 / n)),
        in_specs=[
            hbm_spec,
            hbm_spec,
            hbm_spec,
            pl.BlockSpec((1, n, n_in), lambda: (0, 0, 0)),
            pl.BlockSpec((1, n, n_in), lambda: (0, 0, 0)),
            pl.BlockSpec((h, n_in), lambda: (0, 0)),
            pl.BlockSpec((1, h), lambda: (0, 0)),
            pl.BlockSpec((1, 1), lambda: (0, 0)),
            pl.BlockSpec((1, h, h), lambda: (0, 0, 0)),
            pl.BlockSpec((1, 1), lambda: (0, 0)),
        ],
        out_specs=[
            pl.BlockSpec((nb, bm), lambda: (0, 0)),
            pl.BlockSpec((nb, bm), lambda: (0, 0)),
        ],
        out_shape=[
            jax.ShapeDtypeStruct((nb, bm), jnp.float32),
            jax.ShapeDtypeStruct((nb, bm), jnp.float32),
        ],
        scratch_shapes=[
            pltpu.VMEM((2, bm, n), jnp.float32),
            pltpu.VMEM((n, 2 * h), jnp.bfloat16),
            pltpu.VMEM((1, h), jnp.float32),
            pltpu.VMEM((1, h), jnp.float32),
            pltpu.VMEM((1, h), jnp.float32),
            pltpu.SemaphoreType.DMA((2,)),
        ],
    )(aug_adj1, aug_adj2, adj, seq1, seq2, W_gcn, b2, a2,
      W_bil, b_bil.reshape(1, 1))

    return jnp.concatenate([o1.reshape(1, n), o2.reshape(1, n)], axis=1)
